# baseline (speedup 1.0000x reference)
import functools

import jax
import jax.numpy as jnp
from jax import lax
from jax.experimental import pallas as pl
from jax.experimental.pallas import tpu as pltpu
from jax.experimental.pallas import tpu_sc as plsc

H = 128
LANES = 16
NCORES = 2
NSUB = 16
NW = NCORES * NSUB
CH = 112
SUP = 5


def _sc_message_pass(x, xb, pk, we_in, n_pad, nch0, nch1):
    rows_per_tile = n_pad // NSUB

    mesh = plsc.VectorSubcoreMesh(core_axis_name="c", subcore_axis_name="s",
                                  num_cores=NCORES, num_subcores=NSUB)

    @functools.partial(
        pl.kernel,
        out_type=jax.ShapeDtypeStruct((NCORES, n_pad, H), jnp.float32),
        mesh=mesh,
        scratch_types=[
            pltpu.VMEM((2, SUP, 3, CH), jnp.int32),
            pltpu.VMEM((3, CH, H), jnp.float32),
            pltpu.VMEM((H,), jnp.float32),
            pltpu.VMEM_SHARED((n_pad, H), jnp.float32),
            pltpu.SemaphoreType.DMA,
            pltpu.SemaphoreType.DMA,
            pltpu.SemaphoreType.DMA,
            pltpu.SemaphoreType.DMA,
            pltpu.SemaphoreType.DMA,
            pltpu.SemaphoreType.DMA,
            pltpu.SemaphoreType.DMA,
        ],
    )
    def k(x_hbm, xb_hbm, pk_hbm, we_hbm, out_hbm,
          pk_v, rows_v, we_v, agg_sh, g0, g1, g2, s0, s1, s2, psem):
        c = lax.axis_index("c")
        s = lax.axis_index("s")
        gsem = [g0, g1, g2]
        ssem = [s0, s1, s2]
        cb = jnp.where(c == 0, s * nch0, NSUB * nch0 + s * nch1)
        nch = jnp.where(c == 0, nch0, nch1)

        pltpu.sync_copy(we_hbm, we_v)
        we = [we_v[pl.ds(i * LANES, LANES)] for i in range(H // LANES)]

        row0 = s * rows_per_tile
        pltpu.sync_copy(x_hbm.at[pl.ds(row0, rows_per_tile)],
                        agg_sh.at[pl.ds(row0, rows_per_tile)])
        plsc.subcore_barrier()

        def fire_stage(sup):
            pltpu.async_copy(pk_hbm.at[pl.ds(cb + sup * SUP, SUP)],
                             pk_v.at[lax.rem(sup, 2)], psem)

        def wait_stage(sup):
            pltpu.make_async_copy(pk_hbm.at[pl.ds(cb, SUP)],
                                  pk_v.at[lax.rem(sup, 2)], psem).wait()

        def fire_gather(t, b):
            sup_b = lax.rem(lax.div(t, SUP), 2)
            ch = lax.rem(t, SUP)
            pltpu.async_copy(xb_hbm.at[pk_v.at[sup_b, ch, 0]], rows_v.at[b],
                             gsem[b])

        def wait_gather(b):
            pltpu.make_async_copy(xb_hbm.at[pk_v.at[0, 0, 0]], rows_v.at[b],
                                  gsem[b]).wait()

        def fire_scatter(t, b):
            sup_b = lax.rem(lax.div(t, SUP), 2)
            ch = lax.rem(t, SUP)
            pltpu.async_copy(rows_v.at[b], agg_sh.at[pk_v.at[sup_b, ch, 1]],
                             ssem[b], add=True)

        def drain_scatter(b):
            pltpu.make_async_copy(rows_v.at[b], agg_sh.at[pk_v.at[0, 0, 1]],
                                  ssem[b]).wait()

        fire_stage(0)
        wait_stage(0)
        fire_gather(0, 0)
        fire_gather(1, 1)

        def body(i, carry):
            for j in range(3):
                t = 3 * i + j
                b = j
                wait_gather(b)

                def grp(g, c2):
                    sup_b = lax.rem(lax.div(t, SUP), 2)
                    ch = lax.rem(t, SUP)
                    a16 = lax.bitcast_convert_type(
                        pk_v[sup_b, ch, 2, pl.ds(g * LANES, LANES)],
                        jnp.float32)
                    for kk in range(LANES):
                        rr = g * LANES + kk
                        a = a16.at[jnp.full((LANES,), kk, jnp.int32)].get(
                            mode="promise_in_bounds")
                        for ci in range(H // LANES):
                            v = rows_v[b, rr, pl.ds(ci * LANES, LANES)]
                            v = jnp.maximum(v + a * we[ci], 0.0)
                            rows_v[b, rr, pl.ds(ci * LANES, LANES)] = v
                    return c2

                lax.fori_loop(0, 0, grp, 0)
                fire_scatter(t, b)

                nb = (b + 2) % 3

                @pl.when(t + 2 < nch)
                def _():
                    @pl.when(t >= 1)
                    def _():
                        drain_scatter(nb)

                    @pl.when(jnp.logical_and(lax.rem(t, SUP) == 0,
                                             t < nch - SUP))
                    def _():
                        fire_stage(lax.div(t, SUP) + 1)

                    @pl.when(lax.rem(t + 2, SUP) == 0)
                    def _():
                        wait_stage(lax.div(t + 2, SUP))

                    fire_gather(t + 2, nb)
            return carry

        lax.fori_loop(0, lax.div(nch, 3), body, 0)
        for j in range(3):
            drain_scatter(j)
        plsc.subcore_barrier()
        pltpu.sync_copy(agg_sh.at[pl.ds(row0, rows_per_tile)],
                        out_hbm.at[c, pl.ds(row0, rows_per_tile)])

    return k(x, xb, pk, we_in)


def _tc_proj(nf, wp, bp, bpb, n_pad):
    br = 1024
    grid = (n_pad // br,)

    def body(nf_ref, w_ref, b_ref, bb_ref, o_ref, ob_ref):
        t = jnp.dot(nf_ref[...], w_ref[...], preferred_element_type=jnp.float32)
        o_ref[...] = t + b_ref[...]
        ob_ref[...] = t + bb_ref[...]

    return pl.pallas_call(
        body,
        grid=grid,
        in_specs=[
            pl.BlockSpec((br, H), lambda i: (i, 0)),
            pl.BlockSpec((H, H), lambda i: (0, 0)),
            pl.BlockSpec((1, H), lambda i: (0, 0)),
            pl.BlockSpec((1, H), lambda i: (0, 0)),
        ],
        out_specs=[pl.BlockSpec((br, H), lambda i: (i, 0)),
                   pl.BlockSpec((br, H), lambda i: (i, 0))],
        out_shape=[jax.ShapeDtypeStruct((n_pad, H), jnp.float32),
                   jax.ShapeDtypeStruct((n_pad, H), jnp.float32)],
    )(nf, wp, bp, bpb)


def _tc_mlp(agg2, x, w1, b1, w2, b2, beb, n_pad):
    br = 1024
    grid = (n_pad // br,)

    def body(agg_ref, x_ref, w1_ref, b1_ref, w2_ref, b2_ref, be_ref, o_ref,
             ob_ref):
        hsum = agg_ref[0] + agg_ref[1] - x_ref[...]
        h1 = jnp.maximum(
            jnp.dot(hsum, w1_ref[...], preferred_element_type=jnp.float32)
            + b1_ref[...], 0.0)
        o = jnp.maximum(
            jnp.dot(h1, w2_ref[...], preferred_element_type=jnp.float32)
            + b2_ref[...], 0.0)
        o_ref[...] = o
        ob_ref[...] = o + be_ref[...]

    return pl.pallas_call(
        body,
        grid=grid,
        in_specs=[
            pl.BlockSpec((2, br, H), lambda i: (0, i, 0)),
            pl.BlockSpec((br, H), lambda i: (i, 0)),
            pl.BlockSpec((H, H), lambda i: (0, 0)),
            pl.BlockSpec((1, H), lambda i: (0, 0)),
            pl.BlockSpec((H, H), lambda i: (0, 0)),
            pl.BlockSpec((1, H), lambda i: (0, 0)),
            pl.BlockSpec((1, H), lambda i: (0, 0)),
        ],
        out_specs=[pl.BlockSpec((br, H), lambda i: (i, 0)),
                   pl.BlockSpec((br, H), lambda i: (i, 0))],
        out_shape=[jax.ShapeDtypeStruct((n_pad, H), jnp.float32),
                   jax.ShapeDtypeStruct((n_pad, H), jnp.float32)],
    )(agg2, x, w1, b1, w2, b2, beb)


def kernel(node_features, edge_index, edge_attr, Wp, bp, We, be, W1, b1, W2, b2):
    n, d_in = node_features.shape
    e = edge_index.shape[1]
    nlayers = W1.shape[0]

    n_pad = ((n + NW * LANES - 1) // (NW * LANES)) * (NW * LANES)
    cpw = -(-e // (NW * CH))
    lcm = 3 * SUP // (3 if SUP % 3 == 0 else 1)
    nch = -(-cpw // lcm) * lcm
    e_pad = NW * CH * nch
    nch1 = (2 * nch * 7 // 24 // lcm) * lcm
    nch0 = 2 * nch - nch1

    nf = jnp.pad(node_features.astype(jnp.float32), ((0, n_pad - n), (0, 0)))
    src = jnp.pad(edge_index[0], (0, e_pad - e))
    dst = jnp.pad(edge_index[1], (0, e_pad - e), constant_values=n_pad - 1)
    attr = jnp.pad(edge_attr[:, 0].astype(jnp.float32), (0, e_pad - e))
    pk = jnp.stack([
        src.reshape(e_pad // CH, CH),
        dst.reshape(e_pad // CH, CH),
        lax.bitcast_convert_type(attr, jnp.int32).reshape(e_pad // CH, CH),
    ], axis=1)
    we_row = We[0].astype(jnp.float32)
    be_r = be.reshape(1, H).astype(jnp.float32)

    x, xb = _tc_proj(nf, Wp.astype(jnp.float32), bp.reshape(1, H),
                     bp.reshape(1, H) + be_r, n_pad)
    for l in range(nlayers):
        agg2 = _sc_message_pass(x, xb, pk, we_row, n_pad, nch0, nch1)
        x, xb = _tc_mlp(agg2, x, W1[l], b1[l].reshape(1, H), W2[l],
                        b2[l].reshape(1, H), be_r, n_pad)
    return x[:n]


# speedup vs baseline: 1.0036x; 1.0036x over previous
import functools

import jax
import jax.numpy as jnp
from jax import lax
from jax.experimental import pallas as pl
from jax.experimental.pallas import tpu as pltpu
from jax.experimental.pallas import tpu_sc as plsc

H = 128
LANES = 16
NCORES = 2
NSUB = 16
NW = NCORES * NSUB
CH = 112
SUP = 5


def _sc_message_pass(x, xb, pk, we_in, n_pad, nch0, nch1):
    rows_per_tile = n_pad // NSUB

    mesh = plsc.VectorSubcoreMesh(core_axis_name="c", subcore_axis_name="s",
                                  num_cores=NCORES, num_subcores=NSUB)

    @functools.partial(
        pl.kernel,
        out_type=jax.ShapeDtypeStruct((NCORES, n_pad, H), jnp.float32),
        mesh=mesh,
        scratch_types=[
            pltpu.VMEM((2, SUP, 3, CH), jnp.int32),
            pltpu.VMEM((3, CH, H), jnp.float32),
            pltpu.VMEM((H,), jnp.float32),
            pltpu.VMEM_SHARED((n_pad, H), jnp.float32),
            pltpu.SemaphoreType.DMA,
            pltpu.SemaphoreType.DMA,
            pltpu.SemaphoreType.DMA,
            pltpu.SemaphoreType.DMA,
            pltpu.SemaphoreType.DMA,
            pltpu.SemaphoreType.DMA,
            pltpu.SemaphoreType.DMA,
        ],
    )
    def k(x_hbm, xb_hbm, pk_hbm, we_hbm, out_hbm,
          pk_v, rows_v, we_v, agg_sh, g0, g1, g2, s0, s1, s2, psem):
        c = lax.axis_index("c")
        s = lax.axis_index("s")
        gsem = [g0, g1, g2]
        ssem = [s0, s1, s2]
        cb = jnp.where(c == 0, s * nch0, NSUB * nch0 + s * nch1)
        nch = jnp.where(c == 0, nch0, nch1)

        pltpu.sync_copy(we_hbm, we_v)
        we = [we_v[pl.ds(i * LANES, LANES)] for i in range(H // LANES)]

        row0 = s * rows_per_tile
        pltpu.sync_copy(x_hbm.at[pl.ds(row0, rows_per_tile)],
                        agg_sh.at[pl.ds(row0, rows_per_tile)])
        plsc.subcore_barrier()

        def fire_stage(sup):
            pltpu.async_copy(pk_hbm.at[pl.ds(cb + sup * SUP, SUP)],
                             pk_v.at[lax.rem(sup, 2)], psem)

        def wait_stage(sup):
            pltpu.make_async_copy(pk_hbm.at[pl.ds(cb, SUP)],
                                  pk_v.at[lax.rem(sup, 2)], psem).wait()

        def fire_gather(t, b):
            sup_b = lax.rem(lax.div(t, SUP), 2)
            ch = lax.rem(t, SUP)
            pltpu.async_copy(xb_hbm.at[pk_v.at[sup_b, ch, 0]], rows_v.at[b],
                             gsem[b])

        def wait_gather(b):
            pltpu.make_async_copy(xb_hbm.at[pk_v.at[0, 0, 0]], rows_v.at[b],
                                  gsem[b]).wait()

        def fire_scatter(t, b):
            sup_b = lax.rem(lax.div(t, SUP), 2)
            ch = lax.rem(t, SUP)
            pltpu.async_copy(rows_v.at[b], agg_sh.at[pk_v.at[sup_b, ch, 1]],
                             ssem[b], add=True)

        def drain_scatter(b):
            pltpu.make_async_copy(rows_v.at[b], agg_sh.at[pk_v.at[0, 0, 1]],
                                  ssem[b]).wait()

        fire_stage(0)
        wait_stage(0)
        fire_gather(0, 0)
        fire_gather(1, 1)

        def body(i, carry):
            for j in range(3):
                t = 3 * i + j
                b = j
                wait_gather(b)

                def grp(g, c2):
                    sup_b = lax.rem(lax.div(t, SUP), 2)
                    ch = lax.rem(t, SUP)
                    a16 = lax.bitcast_convert_type(
                        pk_v[sup_b, ch, 2, pl.ds(g * LANES, LANES)],
                        jnp.float32)
                    for kk in range(LANES):
                        rr = g * LANES + kk
                        a = a16.at[jnp.full((LANES,), kk, jnp.int32)].get(
                            mode="promise_in_bounds")
                        for ci in range(H // LANES):
                            v = rows_v[b, rr, pl.ds(ci * LANES, LANES)]
                            v = jnp.maximum(v + a * we[ci], 0.0)
                            rows_v[b, rr, pl.ds(ci * LANES, LANES)] = v
                    return c2

                lax.fori_loop(0, CH // LANES, grp, 0)
                fire_scatter(t, b)

                nb = (b + 2) % 3

                @pl.when(t + 2 < nch)
                def _():
                    @pl.when(t >= 1)
                    def _():
                        drain_scatter(nb)

                    @pl.when(jnp.logical_and(lax.rem(t, SUP) == 0,
                                             t < nch - SUP))
                    def _():
                        fire_stage(lax.div(t, SUP) + 1)

                    @pl.when(lax.rem(t + 2, SUP) == 0)
                    def _():
                        wait_stage(lax.div(t + 2, SUP))

                    fire_gather(t + 2, nb)
            return carry

        lax.fori_loop(0, lax.div(nch, 3), body, 0)
        for j in range(3):
            drain_scatter(j)
        plsc.subcore_barrier()
        pltpu.sync_copy(agg_sh.at[pl.ds(row0, rows_per_tile)],
                        out_hbm.at[c, pl.ds(row0, rows_per_tile)])

    return k(x, xb, pk, we_in)


def _tc_proj(nf, wp, bp, bpb, n_pad):
    br = 1024
    grid = (n_pad // br,)

    def body(nf_ref, w_ref, b_ref, bb_ref, o_ref, ob_ref):
        t = jnp.dot(nf_ref[...], w_ref[...], preferred_element_type=jnp.float32)
        o_ref[...] = t + b_ref[...]
        ob_ref[...] = t + bb_ref[...]

    return pl.pallas_call(
        body,
        grid=grid,
        in_specs=[
            pl.BlockSpec((br, H), lambda i: (i, 0)),
            pl.BlockSpec((H, H), lambda i: (0, 0)),
            pl.BlockSpec((1, H), lambda i: (0, 0)),
            pl.BlockSpec((1, H), lambda i: (0, 0)),
        ],
        out_specs=[pl.BlockSpec((br, H), lambda i: (i, 0)),
                   pl.BlockSpec((br, H), lambda i: (i, 0))],
        out_shape=[jax.ShapeDtypeStruct((n_pad, H), jnp.float32),
                   jax.ShapeDtypeStruct((n_pad, H), jnp.float32)],
    )(nf, wp, bp, bpb)


def _tc_mlp(agg2, x, w1, b1, w2, b2, beb, n_pad):
    br = 1024
    grid = (n_pad // br,)

    def body(agg_ref, x_ref, w1_ref, b1_ref, w2_ref, b2_ref, be_ref, o_ref,
             ob_ref):
        hsum = agg_ref[0] + agg_ref[1] - x_ref[...]
        h1 = jnp.maximum(
            jnp.dot(hsum, w1_ref[...], preferred_element_type=jnp.float32)
            + b1_ref[...], 0.0)
        o = jnp.maximum(
            jnp.dot(h1, w2_ref[...], preferred_element_type=jnp.float32)
            + b2_ref[...], 0.0)
        o_ref[...] = o
        ob_ref[...] = o + be_ref[...]

    return pl.pallas_call(
        body,
        grid=grid,
        in_specs=[
            pl.BlockSpec((2, br, H), lambda i: (0, i, 0)),
            pl.BlockSpec((br, H), lambda i: (i, 0)),
            pl.BlockSpec((H, H), lambda i: (0, 0)),
            pl.BlockSpec((1, H), lambda i: (0, 0)),
            pl.BlockSpec((H, H), lambda i: (0, 0)),
            pl.BlockSpec((1, H), lambda i: (0, 0)),
            pl.BlockSpec((1, H), lambda i: (0, 0)),
        ],
        out_specs=[pl.BlockSpec((br, H), lambda i: (i, 0)),
                   pl.BlockSpec((br, H), lambda i: (i, 0))],
        out_shape=[jax.ShapeDtypeStruct((n_pad, H), jnp.float32),
                   jax.ShapeDtypeStruct((n_pad, H), jnp.float32)],
    )(agg2, x, w1, b1, w2, b2, beb)


def kernel(node_features, edge_index, edge_attr, Wp, bp, We, be, W1, b1, W2, b2):
    n, d_in = node_features.shape
    e = edge_index.shape[1]
    nlayers = W1.shape[0]

    n_pad = ((n + NW * LANES - 1) // (NW * LANES)) * (NW * LANES)
    cpw = -(-e // (NW * CH))
    lcm = 3 * SUP // (3 if SUP % 3 == 0 else 1)
    nch = -(-cpw // lcm) * lcm
    e_pad = NW * CH * nch
    nch1 = (2 * nch * 7 // 24 // lcm) * lcm
    nch0 = 2 * nch - nch1

    nf = jnp.pad(node_features.astype(jnp.float32), ((0, n_pad - n), (0, 0)))
    src = jnp.pad(edge_index[0], (0, e_pad - e))
    dst = jnp.pad(edge_index[1], (0, e_pad - e), constant_values=n_pad - 1)
    attr = jnp.pad(edge_attr[:, 0].astype(jnp.float32), (0, e_pad - e))
    pk = jnp.stack([
        src.reshape(e_pad // CH, CH),
        dst.reshape(e_pad // CH, CH),
        lax.bitcast_convert_type(attr, jnp.int32).reshape(e_pad // CH, CH),
    ], axis=1)
    we_row = We[0].astype(jnp.float32)
    be_r = be.reshape(1, H).astype(jnp.float32)

    x, xb = _tc_proj(nf, Wp.astype(jnp.float32), bp.reshape(1, H),
                     bp.reshape(1, H) + be_r, n_pad)
    for l in range(nlayers):
        agg2 = _sc_message_pass(x, xb, pk, we_row, n_pad, nch0, nch1)
        x, xb = _tc_mlp(agg2, x, W1[l], b1[l].reshape(1, H), W2[l],
                        b2[l].reshape(1, H), be_r, n_pad)
    return x[:n]
